# baseline (speedup 1.0000x reference)
import functools

import jax
import jax.numpy as jnp
from jax import lax
from jax.experimental import pallas as pl
from jax.experimental.pallas import tpu as pltpu
from jax.experimental.pallas import tpu_sc as plsc

N = 32768
H = 128
B = 16
DAPP = 32
NCLS = 2

NC = 2
NS = 16
NW = NC * NS
RW = N // NW
CH = 128
NCH = RW // CH
KS = 5
KV = NCH - KS
L = 16


NR = B + 1


def _seg_sum_body(h_hbm, seg_hbm, tgt_hbm, out_hbm,
                  seg_v, tgt_v, csg2_v, idx16_v, acc_v, buf_s, buf_c,
                  zero_v, acc_sh, gs_sem, gv_sem, ss_sem):
    c = lax.axis_index("c")
    s = lax.axis_index("s")
    wid = s * NC + c
    base = wid * RW

    pltpu.sync_copy(seg_hbm.at[pl.ds(base, RW)], seg_v)
    pltpu.sync_copy(tgt_hbm.at[pl.ds(base, RW)], tgt_v)

    zv = jnp.zeros((L,), jnp.float32)
    lanes = lax.iota(jnp.int32, L)

    @pl.when(s == 0)
    def _zero():
        def zrow(i, carry):
            zero_v[i // (H // L), pl.ds((i % (H // L)) * L, L)] = zv
            return carry

        lax.fori_loop(0, NR * (H // L), zrow, 0)
        pltpu.sync_copy(zero_v, acc_sh)

    for k in range(KS):
        pltpu.async_copy(h_hbm.at[pl.ds(base + k * CH, CH)],
                         buf_s.at[k], gs_sem)
    for k in range(2):
        pltpu.async_copy(h_hbm.at[pl.ds(base + (KS + k) * CH, CH)],
                         buf_c.at[k], gv_sem)

    trash = jnp.zeros((L,), jnp.int32) + B

    def mkidx(i, carry):
        sv = seg_v[pl.ds(i * L, L)]
        tv = tgt_v[pl.ds(i * L, L)]
        csg2_v[i // (CH // L), pl.ds((i % (CH // L)) * L, L)] = jnp.where(
            tv == 1, sv, trash)
        return carry

    lax.fori_loop(0, KS * CH // L, mkidx, 0)
    idx16_v[0, pl.ds(0, L)] = lanes

    def zloc(i, carry):
        acc_v[i // (H // L), pl.ds((i % (H // L)) * L, L)] = zv
        return carry

    lax.fori_loop(0, NR * (H // L), zloc, 0)

    plsc.subcore_barrier()

    scatters = []
    for k in range(KS):
        pltpu.make_async_copy(h_hbm.at[pl.ds(base + k * CH, CH)],
                              buf_s.at[k], gs_sem).wait()
        scatters.append(pltpu.async_copy(
            buf_s.at[k], acc_sh.at[csg2_v.at[k]], ss_sem, add=True))

    for k in range(KV):
        ch = KS + k
        pltpu.make_async_copy(h_hbm.at[pl.ds(base + ch * CH, CH)],
                              buf_c.at[k % 2], gv_sem).wait()

        @plsc.parallel_loop(0, CH // L, 1, unroll=1)
        def grp_body(gb, k=k):
            g0 = gb * L
            sv = seg_v[pl.ds((KS + k) * CH + g0, L)]
            tv = tgt_v[pl.ds((KS + k) * CH + g0, L)]
            rows = jnp.where(tv == 1, sv, B)
            for r in range(L):
                row = rows[r]
                for j in range(H // L):
                    x = buf_c[k % 2, g0 + r, pl.ds(j * L, L)]
                    plsc.addupdate(acc_v.at[row, pl.ds(j * L, L)], x)

        if k + 2 < KV:
            pltpu.async_copy(h_hbm.at[pl.ds(base + (ch + 2) * CH, CH)],
                             buf_c.at[k % 2], gv_sem)

    pltpu.sync_copy(acc_v.at[pl.ds(0, B)], acc_sh.at[idx16_v.at[0]],
                    add=True)
    for d in scatters:
        d.wait()

    plsc.subcore_barrier()

    @pl.when(s == 0)
    def _emit():
        pltpu.sync_copy(acc_sh, out_hbm.at[c])


@functools.lru_cache(maxsize=1)
def _seg_sum():
    return pl.kernel(
        _seg_sum_body,
        out_type=jax.ShapeDtypeStruct((NC, B + 1, H), jnp.float32),
        mesh=plsc.VectorSubcoreMesh(core_axis_name="c", subcore_axis_name="s"),
        scratch_types=[
            pltpu.VMEM((RW,), jnp.int32),
            pltpu.VMEM((RW,), jnp.int32),
            pltpu.VMEM((KS, CH), jnp.int32),
            pltpu.VMEM((1, L), jnp.int32),
            pltpu.VMEM((NR, H), jnp.float32),
            pltpu.VMEM((KS, CH, H), jnp.float32),
            pltpu.VMEM((2, CH, H), jnp.float32),
            pltpu.VMEM((NR, H), jnp.float32),
            pltpu.VMEM_SHARED((NR, H), jnp.float32),
            pltpu.SemaphoreType.DMA,
            pltpu.SemaphoreType.DMA,
            pltpu.SemaphoreType.DMA,
        ],
    )


def _finalize_body(parts_ref, seg_ref, tgt_ref, dep_ref, feat_ref,
                   w1_ref, w2_ref, w3_ref, b_ref, out_ref):
    gh = parts_ref[0, :B, :] + parts_ref[1, :B, :]
    seg = seg_ref[...]
    tgt = tgt_ref[...]
    dep = dep_ref[...]
    gid = lax.broadcasted_iota(jnp.int32, (B,) + seg.shape, 0)
    m = seg[None, :, :] == gid
    num_tot = jnp.sum(m.astype(jnp.float32), axis=(1, 2))
    num_tgt = jnp.sum(jnp.where(jnp.logical_and(m, tgt[None, :, :] == 1),
                                1.0, 0.0), axis=(1, 2))
    mx = jnp.max(jnp.where(m, dep[None, :, :], -jnp.inf), axis=(1, 2))
    logits = (
        jnp.dot(gh, w1_ref[...], preferred_element_type=jnp.float32)
        + jnp.dot(feat_ref[...], w2_ref[...], preferred_element_type=jnp.float32)
        + mx[:, None] * w3_ref[0, :][None, :]
        + num_tgt[:, None] * w3_ref[1, :][None, :]
        + num_tot[:, None] * w3_ref[2, :][None, :]
        + b_ref[0, :][None, :]
    )
    out_ref[...] = logits


def kernel(h, segment_ids, is_target, depth, feature, W, b):
    seg = segment_ids.astype(jnp.int32)
    tgt = is_target.astype(jnp.int32)
    parts = _seg_sum()(h, seg, tgt)
    logits = pl.pallas_call(
        _finalize_body,
        out_shape=jax.ShapeDtypeStruct((B, NCLS), jnp.float32),
    )(parts, seg.reshape(N // H, H), tgt.reshape(N // H, H),
      depth.reshape(N // H, H), feature,
      W[:H], W[H:H + DAPP], W[H + DAPP:], b.reshape(1, NCLS))
    return logits


# speedup vs baseline: 1.2537x; 1.2537x over previous
import functools

import jax
import jax.numpy as jnp
from jax import lax
from jax.experimental import pallas as pl
from jax.experimental.pallas import tpu as pltpu
from jax.experimental.pallas import tpu_sc as plsc

N = 32768
H = 128
B = 16
DAPP = 32
NCLS = 2

NC = 2
NS = 16
NW = NC * NS
RW = N // NW
CH = 128
NCH = RW // CH
NBUF = 4
L = 16


NR = B + 1


def _seg_sum_body(h_hbm, seg_hbm, tgt_hbm, out_hbm,
                  seg_v, tgt_v, csg2_v, buf_v, zero_v,
                  acc_sh, gsem):
    c = lax.axis_index("c")
    s = lax.axis_index("s")
    wid = s * NC + c
    base = wid * RW

    pltpu.sync_copy(seg_hbm.at[pl.ds(base, RW)], seg_v)
    pltpu.sync_copy(tgt_hbm.at[pl.ds(base, RW)], tgt_v)

    zv = jnp.zeros((L,), jnp.float32)
    lanes = lax.iota(jnp.int32, L)

    @pl.when(s == 0)
    def _zero():
        def zrow(i, carry):
            zero_v[i // (H // L), pl.ds((i % (H // L)) * L, L)] = zv
            return carry

        lax.fori_loop(0, NR * (H // L), zrow, 0)
        pltpu.sync_copy(zero_v, acc_sh)

    trash = jnp.zeros((L,), jnp.int32) + B

    def mkidx(i, carry):
        sv = seg_v[pl.ds(i * L, L)]
        tv = tgt_v[pl.ds(i * L, L)]
        csg2_v[i // (CH // L), pl.ds((i % (CH // L)) * L, L)] = jnp.where(
            tv == 1, sv, trash)
        return carry

    lax.fori_loop(0, RW // L, mkidx, 0)

    plsc.subcore_barrier()

    pltpu.async_copy(h_hbm.at[pl.ds(base, CH)], buf_v.at[0], gsem)
    pltpu.async_copy(h_hbm.at[pl.ds(base + CH, CH)], buf_v.at[1], gsem)

    def chunk_body(i, carry):
        slot = lax.rem(i, 2)
        pltpu.make_async_copy(h_hbm.at[pl.ds(base + i * CH, CH)],
                              buf_v.at[slot], gsem).wait()
        pltpu.sync_copy(buf_v.at[slot], acc_sh.at[csg2_v.at[i]], add=True)

        @pl.when(i + 2 < NCH)
        def _next():
            pltpu.async_copy(h_hbm.at[pl.ds(base + (i + 2) * CH, CH)],
                             buf_v.at[slot], gsem)

        return carry

    lax.fori_loop(0, NCH, chunk_body, 0)

    plsc.subcore_barrier()

    @pl.when(s == 0)
    def _emit():
        pltpu.sync_copy(acc_sh, out_hbm.at[c])


@functools.lru_cache(maxsize=1)
def _seg_sum():
    return pl.kernel(
        _seg_sum_body,
        out_type=jax.ShapeDtypeStruct((NC, B + 1, H), jnp.float32),
        mesh=plsc.VectorSubcoreMesh(core_axis_name="c", subcore_axis_name="s"),
        scratch_types=[
            pltpu.VMEM((RW,), jnp.int32),
            pltpu.VMEM((RW,), jnp.int32),
            pltpu.VMEM((NCH, CH), jnp.int32),
            pltpu.VMEM((2, CH, H), jnp.float32),
            pltpu.VMEM((NR, H), jnp.float32),
            pltpu.VMEM_SHARED((NR, H), jnp.float32),
            pltpu.SemaphoreType.DMA,
        ],
    )


def _stats_body(seg_ref, tgt_ref, dep_ref, feat_ref, w2_ref, w3_ref, b_ref,
                out_ref):
    seg = seg_ref[...]
    tgt = tgt_ref[...]
    dep = dep_ref[...]
    gid = lax.broadcasted_iota(jnp.int32, (B,) + seg.shape, 0)
    m = seg[None, :, :] == gid
    num_tot = jnp.sum(m.astype(jnp.float32), axis=(1, 2))
    num_tgt = jnp.sum(jnp.where(jnp.logical_and(m, tgt[None, :, :] == 1),
                                1.0, 0.0), axis=(1, 2))
    mx = jnp.max(jnp.where(m, dep[None, :, :], -jnp.inf), axis=(1, 2))
    out_ref[...] = (
        jnp.dot(feat_ref[...], w2_ref[...], preferred_element_type=jnp.float32)
        + mx[:, None] * w3_ref[0, :][None, :]
        + num_tgt[:, None] * w3_ref[1, :][None, :]
        + num_tot[:, None] * w3_ref[2, :][None, :]
        + b_ref[0, :][None, :]
    )


def _combine_body(parts_ref, rest_ref, w1_ref, out_ref):
    gh = parts_ref[0, :B, :] + parts_ref[1, :B, :]
    out_ref[...] = rest_ref[...] + jnp.dot(
        gh, w1_ref[...], preferred_element_type=jnp.float32)


def kernel(h, segment_ids, is_target, depth, feature, W, b):
    seg = segment_ids.astype(jnp.int32)
    tgt = is_target.astype(jnp.int32)
    parts = _seg_sum()(h, seg, tgt)
    rest = pl.pallas_call(
        _stats_body,
        out_shape=jax.ShapeDtypeStruct((B, NCLS), jnp.float32),
    )(seg.reshape(N // H, H), tgt.reshape(N // H, H),
      depth.reshape(N // H, H), feature,
      W[H:H + DAPP], W[H + DAPP:], b.reshape(1, NCLS))
    logits = pl.pallas_call(
        _combine_body,
        out_shape=jax.ShapeDtypeStruct((B, NCLS), jnp.float32),
    )(parts, rest, W[:H])
    return logits


# speedup vs baseline: 1.3053x; 1.0412x over previous
import functools

import jax
import jax.numpy as jnp
from jax import lax
from jax.experimental import pallas as pl
from jax.experimental.pallas import tpu as pltpu
from jax.experimental.pallas import tpu_sc as plsc

N = 32768
H = 128
B = 16
DAPP = 32
NCLS = 2

NC = 2
NS = 16
NW = NC * NS
RW = N // NW
CH = 128
NCH = RW // CH
NBUF = 4
L = 16


NR = B + 1


def _seg_sum_body(h_hbm, seg_hbm, tgt_hbm, out_hbm,
                  seg_v, tgt_v, csg2_v, idx16_v, acc_v, buf_v, zero_v,
                  acc_sh, gsem):
    c = lax.axis_index("c")
    s = lax.axis_index("s")
    wid = s * NC + c
    base = wid * RW

    pltpu.sync_copy(seg_hbm.at[pl.ds(base, RW)], seg_v)
    pltpu.sync_copy(tgt_hbm.at[pl.ds(base, RW)], tgt_v)

    zv = jnp.zeros((L,), jnp.float32)
    lanes = lax.iota(jnp.int32, L)

    @pl.when(s == 0)
    def _zero():
        def zrow(i, carry):
            zero_v[i // (H // L), pl.ds((i % (H // L)) * L, L)] = zv
            return carry

        lax.fori_loop(0, NR * (H // L), zrow, 0)
        pltpu.sync_copy(zero_v, acc_sh)

    trash = jnp.zeros((L,), jnp.int32) + B

    def mkidx(i, carry):
        sv = seg_v[pl.ds(i * L, L)]
        tv = tgt_v[pl.ds(i * L, L)]
        csg2_v[i // (CH // L), pl.ds((i % (CH // L)) * L, L)] = jnp.where(
            tv == 1, sv, trash)
        return carry

    lax.fori_loop(0, RW // L, mkidx, 0)

    def zloc(i, carry):
        acc_v[i // (H // L), pl.ds((i % (H // L)) * L, L)] = zv
        return carry

    lax.fori_loop(0, B * (H // L), zloc, 0)
    idx16_v[0, pl.ds(0, L)] = lanes

    plsc.subcore_barrier()

    pltpu.async_copy(h_hbm.at[pl.ds(base, CH)], buf_v.at[0], gsem)
    pltpu.async_copy(h_hbm.at[pl.ds(base + CH, CH)], buf_v.at[1], gsem)

    def chunk_body(i, carry):
        slot = lax.rem(i, 2)
        pltpu.make_async_copy(h_hbm.at[pl.ds(base + i * CH, CH)],
                              buf_v.at[slot], gsem).wait()
        svf = seg_v[pl.ds(i * CH, L)]
        svl = seg_v[pl.ds(i * CH + CH - L, L)]
        uni = svf[0] == svl[L - 1]

        @pl.when(uni)
        def _uniform():
            def grp(g, acc):
                g0 = g * L
                tf = tgt_v[pl.ds(i * CH + g0, L)].astype(jnp.float32)
                for r in range(L):
                    mf = tf[r]
                    acc = tuple(
                        acc[j] + buf_v[slot, g0 + r, pl.ds(j * L, L)] * mf
                        for j in range(H // L))
                return acc

            acc0 = tuple(zv for _ in range(H // L))
            accf = lax.fori_loop(0, CH // L, grp, acc0)
            row = svf[0]
            for j in range(H // L):
                plsc.addupdate(acc_v.at[row, pl.ds(j * L, L)], accf[j])

        @pl.when(jnp.logical_not(uni))
        def _mixed():
            pltpu.sync_copy(buf_v.at[slot], acc_sh.at[csg2_v.at[i]],
                            add=True)

        @pl.when(i + 2 < NCH)
        def _next():
            pltpu.async_copy(h_hbm.at[pl.ds(base + (i + 2) * CH, CH)],
                             buf_v.at[slot], gsem)

        return carry

    lax.fori_loop(0, NCH, chunk_body, 0)

    pltpu.sync_copy(acc_v, acc_sh.at[idx16_v.at[0]], add=True)

    plsc.subcore_barrier()

    @pl.when(s == 0)
    def _emit():
        pltpu.sync_copy(acc_sh, out_hbm.at[c])


@functools.lru_cache(maxsize=1)
def _seg_sum():
    return pl.kernel(
        _seg_sum_body,
        out_type=jax.ShapeDtypeStruct((NC, B + 1, H), jnp.float32),
        mesh=plsc.VectorSubcoreMesh(core_axis_name="c", subcore_axis_name="s"),
        scratch_types=[
            pltpu.VMEM((RW,), jnp.int32),
            pltpu.VMEM((RW,), jnp.int32),
            pltpu.VMEM((NCH, CH), jnp.int32),
            pltpu.VMEM((1, L), jnp.int32),
            pltpu.VMEM((B, H), jnp.float32),
            pltpu.VMEM((2, CH, H), jnp.float32),
            pltpu.VMEM((NR, H), jnp.float32),
            pltpu.VMEM_SHARED((NR, H), jnp.float32),
            pltpu.SemaphoreType.DMA,
        ],
    )


def _stats_body(seg_ref, tgt_ref, dep_ref, feat_ref, w2_ref, w3_ref, b_ref,
                out_ref):
    seg = seg_ref[...]
    tgt = tgt_ref[...]
    dep = dep_ref[...]
    gid = lax.broadcasted_iota(jnp.int32, (B,) + seg.shape, 0)
    m = seg[None, :, :] == gid
    num_tot = jnp.sum(m.astype(jnp.float32), axis=(1, 2))
    num_tgt = jnp.sum(jnp.where(jnp.logical_and(m, tgt[None, :, :] == 1),
                                1.0, 0.0), axis=(1, 2))
    mx = jnp.max(jnp.where(m, dep[None, :, :], -jnp.inf), axis=(1, 2))
    out_ref[...] = (
        jnp.dot(feat_ref[...], w2_ref[...], preferred_element_type=jnp.float32)
        + mx[:, None] * w3_ref[0, :][None, :]
        + num_tgt[:, None] * w3_ref[1, :][None, :]
        + num_tot[:, None] * w3_ref[2, :][None, :]
        + b_ref[0, :][None, :]
    )


def _combine_body(parts_ref, rest_ref, w1_ref, out_ref):
    gh = parts_ref[0, :B, :] + parts_ref[1, :B, :]
    out_ref[...] = rest_ref[...] + jnp.dot(
        gh, w1_ref[...], preferred_element_type=jnp.float32)


def kernel(h, segment_ids, is_target, depth, feature, W, b):
    seg = segment_ids.astype(jnp.int32)
    tgt = is_target.astype(jnp.int32)
    parts = _seg_sum()(h, seg, tgt)
    rest = pl.pallas_call(
        _stats_body,
        out_shape=jax.ShapeDtypeStruct((B, NCLS), jnp.float32),
    )(seg.reshape(N // H, H), tgt.reshape(N // H, H),
      depth.reshape(N // H, H), feature,
      W[H:H + DAPP], W[H + DAPP:], b.reshape(1, NCLS))
    logits = pl.pallas_call(
        _combine_body,
        out_shape=jax.ShapeDtypeStruct((B, NCLS), jnp.float32),
    )(parts, rest, W[:H])
    return logits
